# baseline (speedup 1.0000x reference)
import jax
import jax.numpy as jnp
from jax.experimental import pallas as pl
from jax.experimental.pallas import tpu as pltpu

N = 38400
E = 614400
B = 64
P = 15
NPG = 40
EMB = 64
HF = 64
D = 4 * HF


def _prelu(x, a):
    return jnp.where(x > 0, x, a * x)


def _enc_kernel(pooled_ref, w_ref, b_ref, pr_ref, out_ref):
    pooled = pooled_ref[...]
    y = jax.lax.dot_general(pooled, w_ref[...], (((1,), (1,)), ((), ())),
                            preferred_element_type=jnp.float32)
    y = y + b_ref[...]
    out_ref[...] = jnp.where(y > 0, y, pr_ref[...] * y)


def kernel(header_feat, header_edge_index, payload_feat, payload_edge_index, header_mask, payload_mask, params):
    p = params

    def bn(x, g, b):
        m = x.mean(0)
        v = x.var(0)
        return (x - m) / jnp.sqrt(v + 1e-5) * g + b

    def gcn(feat, edges, g):
        src, dst = edges[0], edges[1]
        h = p[g + '_emb'][feat]
        deg = jnp.maximum(jax.ops.segment_sum(jnp.ones(E), dst, num_segments=N), 1.0)[:, None]
        hs = []
        for i in range(4):
            hn = jax.ops.segment_sum(h[src], dst, num_segments=N) / deg
            out = h @ p[f'{g}_Ws{i}'].T + p[f'{g}_bs{i}'] + hn @ p[f'{g}_Wn{i}'].T + p[f'{g}_bn{i}']
            out = _prelu(out, p[f'{g}_pr{i}'][None, :])
            h = bn(out, p[f'{g}_g{i}'], p[f'{g}_b{i}'])
            hs.append(h)
        H4 = jnp.concatenate(hs, -1)
        gid = jnp.arange(N) // NPG
        gm = jax.ops.segment_sum(H4, gid, num_segments=B * P) / NPG
        return gm.reshape(B, P, D)

    hx = gcn(header_feat, header_edge_index, 'h') * header_mask.reshape(B, P, 1)
    px = gcn(payload_feat, payload_edge_index, 'p') * payload_mask.reshape(B, P, 1)

    def filt(x, f):
        y = _prelu(x @ p[f + '_W1'].T + p[f + '_b1'], p[f + '_pr'][None, :, None])
        return y @ p[f + '_W2'].T + p[f + '_b2']

    z1 = jax.nn.sigmoid(filt(hx, 'f1')) * px
    z2 = jax.nn.sigmoid(filt(px, 'f2')) * hx
    s = jnp.concatenate([z1, z2], -1).transpose(1, 0, 2)

    def ldir(x, Wih, Whh, b, rev):
        if rev:
            x = x[::-1]
        Hh = Whh.shape[1]

        def step(c, xt):
            h, cc = c
            gt = xt @ Wih.T + h @ Whh.T + b
            i, f, gg, o = jnp.split(gt, 4, -1)
            cc = jax.nn.sigmoid(f) * cc + jax.nn.sigmoid(i) * jnp.tanh(gg)
            h = jax.nn.sigmoid(o) * jnp.tanh(cc)
            return (h, cc), h

        _, hsq = jax.lax.scan(step, (jnp.zeros((B, Hh)), jnp.zeros((B, Hh))), x)
        return hsq[::-1] if rev else hsq

    x = s
    for l in range(2):
        hf = ldir(x, p[f'lstm_Wih{l}f'], p[f'lstm_Whh{l}f'], p[f'lstm_b{l}f'], False)
        hb = ldir(x, p[f'lstm_Wih{l}b'], p[f'lstm_Whh{l}b'], p[f'lstm_b{l}b'], True)
        x = jnp.concatenate([hf, hb], -1)
    pooled = x.mean(0)

    rep = pl.pallas_call(
        _enc_kernel,
        out_shape=jax.ShapeDtypeStruct((B, D), jnp.float32),
    )(pooled, p['enc_W'], p['enc_b'], jnp.broadcast_to(p['enc_pr'][None, :], (B, D)))
    return rep


# speedup vs baseline: 3.7050x; 3.7050x over previous
import functools

import jax
import jax.numpy as jnp
from jax import lax
from jax.experimental import pallas as pl
from jax.experimental.pallas import tpu as pltpu
from jax.experimental.pallas import tpu_sc as plsc

N = 38400
E = 614400
B = 64
P = 15
NPG = 40
EMB = 64
HF = 64
D = 4 * HF

NCORES = 2
NSUB = 16


def _prelu(x, a):
    return jnp.where(x > 0, x, a * x)


def _make_segsum(n, e, w, ch, rpt):
    nh = n // 2
    accr = NSUB * rpt
    ept = e // NSUB
    nchunk = ept // ch
    assert accr >= nh + 1 and rpt % ch == 0 and ept % ch == 0

    mesh = plsc.VectorSubcoreMesh(core_axis_name="c", subcore_axis_name="s",
                                  num_cores=NCORES, num_subcores=NSUB)

    @functools.partial(
        pl.kernel,
        out_type=jax.ShapeDtypeStruct((NCORES, accr, w), jnp.float32),
        mesh=mesh,
        compiler_params=pltpu.CompilerParams(use_tc_tiling_on_sc=False),
        scratch_types=[
            pltpu.VMEM((ch,), jnp.int32),
            pltpu.VMEM((ch,), jnp.int32),
            pltpu.VMEM((ch, w), jnp.float32),
            pltpu.VMEM((ch, w), jnp.float32),
            pltpu.VMEM_SHARED((accr, w), jnp.float32),
            pltpu.SemaphoreType.DMA,
        ],
    )
    def segsum(h_hbm, src_hbm, dst_hbm, out_hbm, idx_v, ldst_v, rows_v, zero_v,
               acc_sh, sem):
        c = lax.axis_index("c")
        t = lax.axis_index("s")

        z16 = jnp.zeros((16,), jnp.float32)

        def zrow(i, _):
            for j in range(w // 16):
                zero_v[i, pl.ds(j * 16, 16)] = z16
            return 0

        lax.fori_loop(0, ch, zrow, 0)

        row0 = t * rpt

        def zacc(i, _):
            pltpu.sync_copy(zero_v, acc_sh.at[pl.ds(row0 + i * ch, ch), :])
            return 0

        lax.fori_loop(0, rpt // ch, zacc, 0)
        plsc.subcore_barrier()

        base = t * ept
        coff = c * nh

        def chunk(i, _):
            e0 = base + i * ch
            pltpu.sync_copy(src_hbm.at[pl.ds(e0, ch)], idx_v)
            pltpu.sync_copy(dst_hbm.at[pl.ds(e0, ch)], ldst_v)
            for j in range(ch // 16):
                d16 = ldst_v[pl.ds(j * 16, 16)]
                ld = d16 - coff
                oob = (ld < 0) | (ld >= nh)
                ldst_v[pl.ds(j * 16, 16)] = jnp.where(oob, nh, ld)
            pltpu.async_copy(h_hbm.at[idx_v], rows_v, sem).wait()
            pltpu.sync_copy(rows_v, acc_sh.at[ldst_v], add=True)
            return 0

        lax.fori_loop(0, nchunk, chunk, 0)
        plsc.subcore_barrier()

        pltpu.sync_copy(acc_sh.at[pl.ds(row0, rpt), :],
                        out_hbm.at[c, pl.ds(row0, rpt), :])

    return segsum


_segsum_80 = _make_segsum(N, E, 80, 128, 1280)
_segsum_64 = _make_segsum(N, E, 64, 128, 1280)


def kernel(header_feat, header_edge_index, payload_feat, payload_edge_index, header_mask, payload_mask, params):
    p = params
    nh = N // 2

    def bn(x, g, b):
        m = x.mean(0)
        v = x.var(0)
        return (x - m) / jnp.sqrt(v + 1e-5) * g + b

    def segsum(h, src, dst, w):
        f = _segsum_80 if w == 80 else _segsum_64
        out = f(h, src, dst)
        return jnp.concatenate([out[0, :nh], out[1, :nh]], axis=0)

    def gcn(feat, edges, g):
        src, dst = edges[0], edges[1]
        h = p[g + '_emb'][feat]
        h0ext = jnp.concatenate([h, jnp.ones((N, 16), jnp.float32)], axis=1)
        acc0 = segsum(h0ext, src, dst, 80)
        deg = jnp.maximum(acc0[:, 64:65], 1.0)
        hs = []
        for i in range(4):
            if i == 0:
                hn_sum = acc0[:, :64]
            else:
                hn_sum = segsum(h, src, dst, 64)
            hn = hn_sum / deg
            out = h @ p[f'{g}_Ws{i}'].T + p[f'{g}_bs{i}'] + hn @ p[f'{g}_Wn{i}'].T + p[f'{g}_bn{i}']
            out = _prelu(out, p[f'{g}_pr{i}'][None, :])
            h = bn(out, p[f'{g}_g{i}'], p[f'{g}_b{i}'])
            hs.append(h)
        H4 = jnp.concatenate(hs, -1)
        gm = H4.reshape(B * P, NPG, D).mean(1)
        return gm.reshape(B, P, D)

    hx = gcn(header_feat, header_edge_index, 'h') * header_mask.reshape(B, P, 1)
    px = gcn(payload_feat, payload_edge_index, 'p') * payload_mask.reshape(B, P, 1)

    def filt(x, f):
        y = _prelu(x @ p[f + '_W1'].T + p[f + '_b1'], p[f + '_pr'][None, :, None])
        return y @ p[f + '_W2'].T + p[f + '_b2']

    z1 = jax.nn.sigmoid(filt(hx, 'f1')) * px
    z2 = jax.nn.sigmoid(filt(px, 'f2')) * hx
    s = jnp.concatenate([z1, z2], -1).transpose(1, 0, 2)

    def ldir(x, Wih, Whh, b, rev):
        if rev:
            x = x[::-1]
        Hh = Whh.shape[1]

        def step(c, xt):
            h, cc = c
            gt = xt @ Wih.T + h @ Whh.T + b
            i, f, gg, o = jnp.split(gt, 4, -1)
            cc = jax.nn.sigmoid(f) * cc + jax.nn.sigmoid(i) * jnp.tanh(gg)
            h = jax.nn.sigmoid(o) * jnp.tanh(cc)
            return (h, cc), h

        _, hsq = jax.lax.scan(step, (jnp.zeros((B, Hh)), jnp.zeros((B, Hh))), x)
        return hsq[::-1] if rev else hsq

    x = s
    for l in range(2):
        hf = ldir(x, p[f'lstm_Wih{l}f'], p[f'lstm_Whh{l}f'], p[f'lstm_b{l}f'], False)
        hb = ldir(x, p[f'lstm_Wih{l}b'], p[f'lstm_Whh{l}b'], p[f'lstm_b{l}b'], True)
        x = jnp.concatenate([hf, hb], -1)
    pooled = x.mean(0)

    rep = pl.pallas_call(
        _enc_kernel,
        out_shape=jax.ShapeDtypeStruct((B, D), jnp.float32),
    )(pooled, p['enc_W'], p['enc_b'], jnp.broadcast_to(p['enc_pr'][None, :], (B, D)))
    return rep


def _enc_kernel(pooled_ref, w_ref, b_ref, pr_ref, out_ref):
    pooled = pooled_ref[...]
    y = lax.dot_general(pooled, w_ref[...], (((1,), (1,)), ((), ())),
                        preferred_element_type=jnp.float32)
    y = y + b_ref[...]
    out_ref[...] = jnp.where(y > 0, y, pr_ref[...] * y)


# speedup vs baseline: 6.2372x; 1.6835x over previous
import functools

import jax
import jax.numpy as jnp
from jax import lax
from jax.experimental import pallas as pl
from jax.experimental.pallas import tpu as pltpu
from jax.experimental.pallas import tpu_sc as plsc

N = 38400
E = 614400
B = 64
P = 15
NPG = 40
EMB = 64
HF = 64
D = 4 * HF

NCORES = 2
NSUB = 16


def _prelu(x, a):
    return jnp.where(x > 0, x, a * x)


def _make_edgepart(n, e, cap, blk):
    nh = n // 2
    ept = e // NSUB
    nblk_in = ept // blk
    assert ept % blk == 0
    mesh = plsc.VectorSubcoreMesh(core_axis_name="c", subcore_axis_name="s",
                                  num_cores=NCORES, num_subcores=NSUB)

    @functools.partial(
        pl.kernel,
        out_type=(
            jax.ShapeDtypeStruct((NCORES, NSUB, cap), jnp.int32),
            jax.ShapeDtypeStruct((NCORES, NSUB, cap), jnp.int32),
            jax.ShapeDtypeStruct((NCORES, NSUB, 16), jnp.int32),
        ),
        mesh=mesh,
        compiler_params=pltpu.CompilerParams(use_tc_tiling_on_sc=False, needs_layout_passes=False),
        scratch_types=[
            pltpu.VMEM((blk,), jnp.int32),
            pltpu.VMEM((blk,), jnp.int32),
            pltpu.VMEM((cap,), jnp.int32),
            pltpu.VMEM((cap,), jnp.int32),
            pltpu.VMEM((16,), jnp.int32),
        ],
    )
    def edgepart(src_hbm, dst_hbm, csrc_hbm, cldst_hbm, cnt_hbm,
                 srcb, dstb, ssrc, sldst, scnt):
        c = lax.axis_index("c")
        t = lax.axis_index("s")
        coff = c * nh
        base = t * ept
        zi16 = jnp.zeros((16,), jnp.int32)
        dum16 = jnp.full((16,), nh, jnp.int32)

        def scan_blk(bi, cnt):
            pltpu.sync_copy(src_hbm.at[pl.ds(base + bi * blk, blk)], srcb)
            pltpu.sync_copy(dst_hbm.at[pl.ds(base + bi * blk, blk)], dstb)

            lanes = lax.broadcasted_iota(jnp.int32, (16,), 0)

            def grp(k, cn):
                sv = srcb[pl.ds(k * 16, 16)]
                dv = dstb[pl.ds(k * 16, 16)]
                ld = dv - coff
                m = (ld >= 0) & (ld < nh)
                incl = plsc.cumsum(
                    lax.select(m, jnp.full((16,), 1, jnp.int32),
                               jnp.zeros((16,), jnp.int32)))
                pos = jnp.where(m, cn + incl - 1, cap - 16 + lanes)
                plsc.store_scatter(ssrc, [pos], sv)
                plsc.store_scatter(sldst, [pos], jnp.where(m, ld, nh))
                return cn + jnp.max(incl)

            return lax.fori_loop(0, blk // 16, grp, cnt)

        cnt = lax.fori_loop(0, nblk_in, scan_blk, jnp.int32(0))

        nout = (cnt + 512 + 2047) // 2048

        lanes0 = lax.broadcasted_iota(jnp.int32, (16,), 0)

        def fill(k, _):
            pos = jnp.minimum(cnt + k * 16 + lanes0, cap - 16 + lanes0)
            plsc.store_scatter(ssrc, [pos], zi16)
            plsc.store_scatter(sldst, [pos], dum16)
            return 0

        lax.fori_loop(0, (nout * 2048 - cnt + 15) // 16, fill, 0)

        def wb(bi, _):
            pltpu.sync_copy(ssrc.at[pl.ds(bi * 2048, 2048)],
                            csrc_hbm.at[c, t, pl.ds(bi * 2048, 2048)])
            pltpu.sync_copy(sldst.at[pl.ds(bi * 2048, 2048)],
                            cldst_hbm.at[c, t, pl.ds(bi * 2048, 2048)])
            return 0

        lax.fori_loop(0, nout, wb, 0)

        nch = (cnt + 127) // 128
        scnt[pl.ds(0, 16)] = jnp.broadcast_to(nch, (16,)).astype(jnp.int32)
        pltpu.sync_copy(scnt, cnt_hbm.at[c, t, :])

    return edgepart


def _make_segsum2(n, e, w, cap, rpt):
    nh = n // 2
    accr = NSUB * rpt
    ch = 128
    mesh = plsc.VectorSubcoreMesh(core_axis_name="c", subcore_axis_name="s",
                                  num_cores=NCORES, num_subcores=NSUB)

    @functools.partial(
        pl.kernel,
        out_type=jax.ShapeDtypeStruct((NCORES, accr, w), jnp.float32),
        mesh=mesh,
        compiler_params=pltpu.CompilerParams(use_tc_tiling_on_sc=False, needs_layout_passes=False),
        scratch_types=[
            pltpu.VMEM((ch,), jnp.int32),
            pltpu.VMEM((ch,), jnp.int32),
            pltpu.VMEM((ch, w), jnp.float32),
            pltpu.VMEM((ch,), jnp.int32),
            pltpu.VMEM((ch,), jnp.int32),
            pltpu.VMEM((ch, w), jnp.float32),
            pltpu.VMEM((64, w), jnp.float32),
            pltpu.VMEM((16,), jnp.int32),
            pltpu.VMEM_SHARED((accr, w), jnp.float32),
            pltpu.SemaphoreType.DMA,
            pltpu.SemaphoreType.DMA,
        ],
    )
    def segsum2(h_hbm, csrc_hbm, cldst_hbm, cnt_hbm, out_hbm,
                idxa, lda, rowsa, idxb, ldb, rowsb, zero_v, cntv, acc_sh,
                sema, semb):
        c = lax.axis_index("c")
        t = lax.axis_index("s")
        z16 = jnp.zeros((16,), jnp.float32)

        def zrow(i, _):
            for j in range(w // 16):
                zero_v[i, pl.ds(j * 16, 16)] = z16
            return 0

        lax.fori_loop(0, 64, zrow, 0)
        row0 = t * rpt

        def zacc(i, _):
            pltpu.sync_copy(zero_v, acc_sh.at[pl.ds(row0 + i * 64, 64), :])
            return 0

        lax.fori_loop(0, rpt // 64, zacc, 0)

        pltpu.sync_copy(cnt_hbm.at[c, t, :], cntv)
        nch = jnp.max(cntv[pl.ds(0, 16)])
        nch2 = (nch + 1) // 2
        plsc.subcore_barrier()

        def load_idx(k, iv, lv):
            pltpu.sync_copy(csrc_hbm.at[c, t, pl.ds(k * ch, ch)], iv)
            pltpu.sync_copy(cldst_hbm.at[c, t, pl.ds(k * ch, ch)], lv)

        load_idx(0, idxa, lda)
        pltpu.async_copy(h_hbm.at[idxa], rowsa, sema)

        def pipe(j, _):
            load_idx(2 * j + 1, idxb, ldb)
            pltpu.async_copy(h_hbm.at[idxb], rowsb, semb)
            pltpu.make_async_copy(h_hbm.at[idxa], rowsa, sema).wait()
            pltpu.sync_copy(rowsa, acc_sh.at[lda], add=True)
            load_idx(2 * j + 2, idxa, lda)
            pltpu.async_copy(h_hbm.at[idxa], rowsa, sema)
            pltpu.make_async_copy(h_hbm.at[idxb], rowsb, semb).wait()
            pltpu.sync_copy(rowsb, acc_sh.at[ldb], add=True)
            return 0

        lax.fori_loop(0, nch2, pipe, 0)
        pltpu.make_async_copy(h_hbm.at[idxa], rowsa, sema).wait()
        pltpu.sync_copy(rowsa, acc_sh.at[lda], add=True)
        plsc.subcore_barrier()

        pltpu.sync_copy(acc_sh.at[pl.ds(row0, rpt), :],
                        out_hbm.at[c, pl.ds(row0, rpt), :])

    return segsum2


_CAP = 40960
_edgepart = _make_edgepart(N, E, _CAP, 9600)
_segsum2_80 = _make_segsum2(N, E, 80, _CAP, 1216)
_segsum2_64 = _make_segsum2(N, E, 64, _CAP, 1216)


def kernel(header_feat, header_edge_index, payload_feat, payload_edge_index, header_mask, payload_mask, params):
    p = params
    nh = N // 2

    def bn(x, g, b):
        m = x.mean(0)
        v = x.var(0)
        return (x - m) / jnp.sqrt(v + 1e-5) * g + b

    def gcn(feat, edges, g):
        src, dst = edges[0], edges[1]
        csrc, cldst, cnts = _edgepart(src, dst)

        def segsum(h, w):
            f = _segsum2_80 if w == 80 else _segsum2_64
            out = f(h, csrc, cldst, cnts)
            return jnp.concatenate([out[0, :nh], out[1, :nh]], axis=0)

        h = p[g + '_emb'][feat]
        h0ext = jnp.concatenate([h, jnp.ones((N, 16), jnp.float32)], axis=1)
        acc0 = segsum(h0ext, 80)
        deg = jnp.maximum(acc0[:, 64:65], 1.0)
        hs = []
        for i in range(4):
            if i == 0:
                hn_sum = acc0[:, :64]
            else:
                hn_sum = segsum(h, 64)
            hn = hn_sum / deg
            out = h @ p[f'{g}_Ws{i}'].T + p[f'{g}_bs{i}'] + hn @ p[f'{g}_Wn{i}'].T + p[f'{g}_bn{i}']
            out = _prelu(out, p[f'{g}_pr{i}'][None, :])
            h = bn(out, p[f'{g}_g{i}'], p[f'{g}_b{i}'])
            hs.append(h)
        H4 = jnp.concatenate(hs, -1)
        gm = H4.reshape(B * P, NPG, D).mean(1)
        return gm.reshape(B, P, D)

    hx = gcn(header_feat, header_edge_index, 'h') * header_mask.reshape(B, P, 1)
    px = gcn(payload_feat, payload_edge_index, 'p') * payload_mask.reshape(B, P, 1)

    def filt(x, f):
        y = _prelu(x @ p[f + '_W1'].T + p[f + '_b1'], p[f + '_pr'][None, :, None])
        return y @ p[f + '_W2'].T + p[f + '_b2']

    z1 = jax.nn.sigmoid(filt(hx, 'f1')) * px
    z2 = jax.nn.sigmoid(filt(px, 'f2')) * hx
    s = jnp.concatenate([z1, z2], -1).transpose(1, 0, 2)

    def ldir(x, Wih, Whh, b, rev):
        if rev:
            x = x[::-1]
        Hh = Whh.shape[1]

        def step(c, xt):
            h, cc = c
            gt = xt @ Wih.T + h @ Whh.T + b
            i, f, gg, o = jnp.split(gt, 4, -1)
            cc = jax.nn.sigmoid(f) * cc + jax.nn.sigmoid(i) * jnp.tanh(gg)
            h = jax.nn.sigmoid(o) * jnp.tanh(cc)
            return (h, cc), h

        _, hsq = jax.lax.scan(step, (jnp.zeros((B, Hh)), jnp.zeros((B, Hh))), x)
        return hsq[::-1] if rev else hsq

    x = s
    for l in range(2):
        hf = ldir(x, p[f'lstm_Wih{l}f'], p[f'lstm_Whh{l}f'], p[f'lstm_b{l}f'], False)
        hb = ldir(x, p[f'lstm_Wih{l}b'], p[f'lstm_Whh{l}b'], p[f'lstm_b{l}b'], True)
        x = jnp.concatenate([hf, hb], -1)
    pooled = x.mean(0)

    rep = pl.pallas_call(
        _enc_kernel,
        out_shape=jax.ShapeDtypeStruct((B, D), jnp.float32),
    )(pooled, p['enc_W'], p['enc_b'], jnp.broadcast_to(p['enc_pr'][None, :], (B, D)))
    return rep


def _enc_kernel(pooled_ref, w_ref, b_ref, pr_ref, out_ref):
    pooled = pooled_ref[...]
    y = lax.dot_general(pooled, w_ref[...], (((1,), (1,)), ((), ())),
                        preferred_element_type=jnp.float32)
    y = y + b_ref[...]
    out_ref[...] = jnp.where(y > 0, y, pr_ref[...] * y)


# speedup vs baseline: 6.6358x; 1.0639x over previous
import functools

import jax
import jax.numpy as jnp
from jax import lax
from jax.experimental import pallas as pl
from jax.experimental.pallas import tpu as pltpu
from jax.experimental.pallas import tpu_sc as plsc

N = 38400
E = 614400
B = 64
P = 15
NPG = 40
EMB = 64
HF = 64
D = 4 * HF

NCORES = 2
NSUB = 16


def _prelu(x, a):
    return jnp.where(x > 0, x, a * x)


def _make_edgepart(n, e, cap, blk):
    nh = n // 2
    ept = e // NSUB
    nblk_in = ept // blk
    assert ept % blk == 0
    mesh = plsc.VectorSubcoreMesh(core_axis_name="c", subcore_axis_name="s",
                                  num_cores=NCORES, num_subcores=NSUB)

    @functools.partial(
        pl.kernel,
        out_type=(
            jax.ShapeDtypeStruct((NCORES, NSUB, cap), jnp.int32),
            jax.ShapeDtypeStruct((NCORES, NSUB, cap), jnp.int32),
            jax.ShapeDtypeStruct((NCORES, NSUB, 16), jnp.int32),
        ),
        mesh=mesh,
        compiler_params=pltpu.CompilerParams(use_tc_tiling_on_sc=False, needs_layout_passes=False),
        scratch_types=[
            pltpu.VMEM((blk,), jnp.int32),
            pltpu.VMEM((blk,), jnp.int32),
            pltpu.VMEM((cap,), jnp.int32),
            pltpu.VMEM((cap,), jnp.int32),
            pltpu.VMEM((16,), jnp.int32),
        ],
    )
    def edgepart(src_hbm, dst_hbm, csrc_hbm, cldst_hbm, cnt_hbm,
                 srcb, dstb, ssrc, sldst, scnt):
        c = lax.axis_index("c")
        t = lax.axis_index("s")
        coff = c * nh
        base = t * ept
        zi16 = jnp.zeros((16,), jnp.int32)
        dum16 = jnp.full((16,), nh, jnp.int32)

        def scan_blk(bi, cnt):
            pltpu.sync_copy(src_hbm.at[pl.ds(base + bi * blk, blk)], srcb)
            pltpu.sync_copy(dst_hbm.at[pl.ds(base + bi * blk, blk)], dstb)

            lanes = lax.broadcasted_iota(jnp.int32, (16,), 0)

            def grp(k, cn):
                sv = srcb[pl.ds(k * 16, 16)]
                dv = dstb[pl.ds(k * 16, 16)]
                ld = dv - coff
                m = (ld >= 0) & (ld < nh)
                incl = plsc.cumsum(
                    lax.select(m, jnp.full((16,), 1, jnp.int32),
                               jnp.zeros((16,), jnp.int32)))
                pos = jnp.where(m, cn + incl - 1, cap - 16 + lanes)
                plsc.store_scatter(ssrc, [pos], sv)
                plsc.store_scatter(sldst, [pos], jnp.where(m, ld, nh))
                return cn + jnp.max(incl)

            return lax.fori_loop(0, blk // 16, grp, cnt)

        cnt = lax.fori_loop(0, nblk_in, scan_blk, jnp.int32(0))

        nout = (cnt + 512 + 2047) // 2048

        lanes0 = lax.broadcasted_iota(jnp.int32, (16,), 0)

        def fill(k, _):
            pos = jnp.minimum(cnt + k * 16 + lanes0, cap - 16 + lanes0)
            plsc.store_scatter(ssrc, [pos], zi16)
            plsc.store_scatter(sldst, [pos], dum16)
            return 0

        lax.fori_loop(0, (nout * 2048 - cnt + 15) // 16, fill, 0)

        def wb(bi, _):
            pltpu.sync_copy(ssrc.at[pl.ds(bi * 2048, 2048)],
                            csrc_hbm.at[c, t, pl.ds(bi * 2048, 2048)])
            pltpu.sync_copy(sldst.at[pl.ds(bi * 2048, 2048)],
                            cldst_hbm.at[c, t, pl.ds(bi * 2048, 2048)])
            return 0

        lax.fori_loop(0, nout, wb, 0)

        nch = (cnt + 127) // 128
        scnt[pl.ds(0, 16)] = jnp.broadcast_to(nch, (16,)).astype(jnp.int32)
        pltpu.sync_copy(scnt, cnt_hbm.at[c, t, :])

    return edgepart


def _make_segsum2(n, e, w, cap, rpt):
    nh = n // 2
    accr = NSUB * rpt
    ch = 128
    mesh = plsc.VectorSubcoreMesh(core_axis_name="c", subcore_axis_name="s",
                                  num_cores=NCORES, num_subcores=NSUB)

    @functools.partial(
        pl.kernel,
        out_type=jax.ShapeDtypeStruct((NCORES, accr, w), jnp.float32),
        mesh=mesh,
        compiler_params=pltpu.CompilerParams(use_tc_tiling_on_sc=False, needs_layout_passes=False),
        scratch_types=[
            pltpu.VMEM((ch,), jnp.int32),
            pltpu.VMEM((ch,), jnp.int32),
            pltpu.VMEM((ch, w), jnp.float32),
            pltpu.VMEM((ch,), jnp.int32),
            pltpu.VMEM((ch,), jnp.int32),
            pltpu.VMEM((ch, w), jnp.float32),
            pltpu.VMEM((64, w), jnp.float32),
            pltpu.VMEM((16,), jnp.int32),
            pltpu.VMEM_SHARED((accr, w), jnp.float32),
            pltpu.SemaphoreType.DMA,
            pltpu.SemaphoreType.DMA,
        ],
    )
    def segsum2(h_hbm, csrc_hbm, cldst_hbm, cnt_hbm, out_hbm,
                idxa, lda, rowsa, idxb, ldb, rowsb, zero_v, cntv, acc_sh,
                sema, semb):
        c = lax.axis_index("c")
        t = lax.axis_index("s")
        z16 = jnp.zeros((16,), jnp.float32)

        def zrow(i, _):
            for j in range(w // 16):
                zero_v[i, pl.ds(j * 16, 16)] = z16
            return 0

        lax.fori_loop(0, 64, zrow, 0)
        row0 = t * rpt

        def zacc(i, _):
            pltpu.sync_copy(zero_v, acc_sh.at[pl.ds(row0 + i * 64, 64), :])
            return 0

        lax.fori_loop(0, rpt // 64, zacc, 0)

        pltpu.sync_copy(cnt_hbm.at[c, t, :], cntv)
        nch = jnp.max(cntv[pl.ds(0, 16)])
        nch2 = (nch + 1) // 2
        plsc.subcore_barrier()

        def load_idx(k, iv, lv):
            pltpu.sync_copy(csrc_hbm.at[c, t, pl.ds(k * ch, ch)], iv)
            pltpu.sync_copy(cldst_hbm.at[c, t, pl.ds(k * ch, ch)], lv)

        load_idx(0, idxa, lda)
        pltpu.async_copy(h_hbm.at[idxa], rowsa, sema)

        def pipe(j, _):
            load_idx(2 * j + 1, idxb, ldb)
            pltpu.async_copy(h_hbm.at[idxb], rowsb, semb)
            pltpu.make_async_copy(h_hbm.at[idxa], rowsa, sema).wait()
            pltpu.sync_copy(rowsa, acc_sh.at[lda], add=True)
            load_idx(2 * j + 2, idxa, lda)
            pltpu.async_copy(h_hbm.at[idxa], rowsa, sema)
            pltpu.make_async_copy(h_hbm.at[idxb], rowsb, semb).wait()
            pltpu.sync_copy(rowsb, acc_sh.at[ldb], add=True)
            return 0

        lax.fori_loop(0, nch2, pipe, 0)
        pltpu.make_async_copy(h_hbm.at[idxa], rowsa, sema).wait()
        pltpu.sync_copy(rowsa, acc_sh.at[lda], add=True)
        plsc.subcore_barrier()

        pltpu.sync_copy(acc_sh.at[pl.ds(row0, rpt), :],
                        out_hbm.at[c, pl.ds(row0, rpt), :])

    return segsum2


_CAP = 40960
_edgepart = _make_edgepart(N, E, _CAP, 9600)
_segsum2_80 = _make_segsum2(N, E, 80, _CAP, 1216)
_segsum2_64 = _make_segsum2(N, E, 64, _CAP, 1216)


_RB_EMB = 640
_RB_D = 768
_RB_N = 320


def _emb_body(feat_ref, emb_ref, out_ref):
    f = feat_ref[...]
    cols = lax.broadcasted_iota(jnp.int32, (1, 320), 1)
    oh = (f == cols).astype(jnp.float32)
    out_ref[:, :64] = lax.dot_general(oh, emb_ref[...], (((1,), (0,)), ((), ())),
                                      preferred_element_type=jnp.float32)
    out_ref[:, 64:80] = jnp.ones((_RB_EMB, 16), jnp.float32)


def _emb_call(feat, emb_pad):
    return pl.pallas_call(
        _emb_body,
        grid=(N // _RB_EMB,),
        in_specs=[
            pl.BlockSpec((_RB_EMB, 1), lambda i: (i, 0)),
            pl.BlockSpec((320, 64), lambda i: (0, 0)),
        ],
        out_specs=pl.BlockSpec((_RB_EMB, 80), lambda i: (i, 0)),
        out_shape=jax.ShapeDtypeStruct((N, 80), jnp.float32),
    )(feat.reshape(N, 1).astype(jnp.int32), emb_pad)


def _dense_body(h_ref, hn_ref, deg_ref, wst_ref, wnt_ref, b_ref, pr_ref,
                act_ref, stats_ref):
    i = pl.program_id(0)
    hm = h_ref[...]
    inv = 1.0 / jnp.maximum(deg_ref[0][:, 64:65], 1.0)
    hn = hn_ref[0][:, 0:64] * inv
    pre = (lax.dot_general(hm, wst_ref[...], (((1,), (0,)), ((), ())),
                           preferred_element_type=jnp.float32)
           + lax.dot_general(hn, wnt_ref[...], (((1,), (0,)), ((), ())),
                             preferred_element_type=jnp.float32)
           + b_ref[...])
    act = jnp.where(pre > 0, pre, pr_ref[...] * pre)
    act_ref[...] = act

    @pl.when(i == 0)
    def _():
        stats_ref[...] = jnp.zeros((16, 64), jnp.float32)

    sm = jnp.sum(act, axis=0, keepdims=True)
    sq = jnp.sum(act * act, axis=0, keepdims=True)
    stats_ref[0:8, :] += jnp.broadcast_to(sm, (8, 64))
    stats_ref[8:16, :] += jnp.broadcast_to(sq, (8, 64))


def _dense_call(h, seg, seg80, wst, wnt, b, pr):
    wseg = seg.shape[2]
    return pl.pallas_call(
        _dense_body,
        grid=(N // _RB_D,),
        in_specs=[
            pl.BlockSpec((_RB_D, 64), lambda i: (i, 0)),
            pl.BlockSpec((1, _RB_D, wseg), lambda i: (i // 25, i % 25, 0)),
            pl.BlockSpec((1, _RB_D, 80), lambda i: (i // 25, i % 25, 0)),
            pl.BlockSpec((64, 64), lambda i: (0, 0)),
            pl.BlockSpec((64, 64), lambda i: (0, 0)),
            pl.BlockSpec((1, 64), lambda i: (0, 0)),
            pl.BlockSpec((1, 64), lambda i: (0, 0)),
        ],
        out_specs=[
            pl.BlockSpec((_RB_D, 64), lambda i: (i, 0)),
            pl.BlockSpec((16, 64), lambda i: (0, 0)),
        ],
        out_shape=[
            jax.ShapeDtypeStruct((N, 64), jnp.float32),
            jax.ShapeDtypeStruct((16, 64), jnp.float32),
        ],
    )(h, seg, seg80, wst, wnt, b, pr)


def _norm_body(act_ref, stats_ref, g_ref, b_ref, h_ref, gm_ref):
    inv_n = 1.0 / N
    m = stats_ref[0:1, :] * inv_n
    ex2 = stats_ref[8:9, :] * inv_n
    var = ex2 - m * m
    scale = g_ref[...] * lax.rsqrt(var + 1e-5)
    hnew = (act_ref[...] - m) * scale + b_ref[...]
    h_ref[...] = hnew
    r8 = lax.broadcasted_iota(jnp.int32, (8, _RB_N), 0)
    c8 = lax.broadcasted_iota(jnp.int32, (8, _RB_N), 1)
    pm = jnp.where(c8 // NPG == r8, 1.0 / NPG, 0.0)
    gm_ref[...] = lax.dot_general(pm, hnew, (((1,), (0,)), ((), ())),
                                  preferred_element_type=jnp.float32)


def _norm_call(act, stats, g, b):
    return pl.pallas_call(
        _norm_body,
        grid=(N // _RB_N,),
        in_specs=[
            pl.BlockSpec((_RB_N, 64), lambda i: (i, 0)),
            pl.BlockSpec((16, 64), lambda i: (0, 0)),
            pl.BlockSpec((1, 64), lambda i: (0, 0)),
            pl.BlockSpec((1, 64), lambda i: (0, 0)),
        ],
        out_specs=[
            pl.BlockSpec((_RB_N, 64), lambda i: (i, 0)),
            pl.BlockSpec((8, 64), lambda i: (i, 0)),
        ],
        out_shape=[
            jax.ShapeDtypeStruct((N, 64), jnp.float32),
            jax.ShapeDtypeStruct((B * P, 64), jnp.float32),
        ],
    )(act, stats, g, b)


def _tail_body(gmh_ref, gmp_ref, hm_ref, pm_ref,
               f1w1_ref, f1b1_ref, f1pr_ref, f1w2_ref, f1b2_ref,
               f2w1_ref, f2b1_ref, f2pr_ref, f2w2_ref, f2b2_ref,
               wih0f_ref, whh0f_ref, b0f_ref, wih0b_ref, whh0b_ref, b0b_ref,
               wih1f_ref, whh1f_ref, b1f_ref, wih1b_ref, whh1b_ref, b1b_ref,
               encw_ref, encb_ref, encpr_ref, out_ref):
    def mm(a, bref):
        return lax.dot_general(a, bref[...], (((1,), (0,)), ((), ())),
                               preferred_element_type=jnp.float32)

    hx = gmh_ref[...] * hm_ref[...]
    px = gmp_ref[...] * pm_ref[...]

    def filt(x, w1, b1, pr, w2, b2):
        y = mm(x, w1) + b1[...]
        y = jnp.where(y > 0, y, pr[...] * y)
        return mm(y, w2) + b2[...]

    z1 = jax.nn.sigmoid(filt(hx, f1w1_ref, f1b1_ref, f1pr_ref, f1w2_ref,
                             f1b2_ref)) * px
    z2 = jax.nn.sigmoid(filt(px, f2w1_ref, f2b1_ref, f2pr_ref, f2w2_ref,
                             f2b2_ref)) * hx
    s3 = jnp.concatenate([z1, z2], axis=1).reshape(B, P, 2 * D)

    def lstm(get_x, wih, whh, bias, rev):
        hstate = jnp.zeros((B, 512), jnp.float32)
        cstate = jnp.zeros((B, 512), jnp.float32)
        outs = [None] * P
        for tt in range(P):
            t = P - 1 - tt if rev else tt
            gt = mm(get_x(t), wih) + mm(hstate, whh) + bias[...]
            ii = jax.nn.sigmoid(gt[:, 0:512])
            ff = jax.nn.sigmoid(gt[:, 512:1024])
            gg = jnp.tanh(gt[:, 1024:1536])
            oo = jax.nn.sigmoid(gt[:, 1536:2048])
            cstate = ff * cstate + ii * gg
            hstate = oo * jnp.tanh(cstate)
            outs[t] = hstate
        return outs

    x0 = lambda t: s3[:, t, :]
    hf = lstm(x0, wih0f_ref, whh0f_ref, b0f_ref, False)
    hb = lstm(x0, wih0b_ref, whh0b_ref, b0b_ref, True)
    x1l = [jnp.concatenate([hf[t], hb[t]], axis=1) for t in range(P)]
    x1 = lambda t: x1l[t]
    hf2 = lstm(x1, wih1f_ref, whh1f_ref, b1f_ref, False)
    hb2 = lstm(x1, wih1b_ref, whh1b_ref, b1b_ref, True)
    pooled = sum(jnp.concatenate([hf2[t], hb2[t]], axis=1) for t in range(P))
    pooled = pooled * (1.0 / P)
    rep = mm(pooled, encw_ref) + encb_ref[...]
    out_ref[...] = jnp.where(rep > 0, rep, encpr_ref[...] * rep)


def _tail_call(gmh, gmp, hmask_b, pmask_b, args):
    return pl.pallas_call(
        _tail_body,
        in_specs=[
            pl.BlockSpec(a.shape, lambda i, r=len(a.shape): (0,) * r)
            for a in (gmh, gmp, hmask_b, pmask_b, *args)],
        out_specs=pl.BlockSpec((B, D), lambda i: (0, 0)),
        out_shape=jax.ShapeDtypeStruct((B, D), jnp.float32),
        grid=(1,),
        compiler_params=pltpu.CompilerParams(
            vmem_limit_bytes=120 * 1024 * 1024),
    )(gmh, gmp, hmask_b, pmask_b, *args)


def kernel(header_feat, header_edge_index, payload_feat, payload_edge_index, header_mask, payload_mask, params):
    p = params
    nh = N // 2

    def gcn(feat, edges, g):
        src, dst = edges[0], edges[1]
        csrc, cldst, cnts = _edgepart(src, dst)
        emb_pad = jnp.pad(p[g + '_emb'], ((0, 320 - 257), (0, 0)))
        h = _emb_call(feat, emb_pad)
        seg80 = _segsum2_80(h, csrc, cldst, cnts)
        gms = []
        for i in range(4):
            if i == 0:
                seg = seg80
            else:
                seg = _segsum2_64(h, csrc, cldst, cnts)
            wst = p[f'{g}_Ws{i}'].T
            wnt = p[f'{g}_Wn{i}'].T
            b = (p[f'{g}_bs{i}'] + p[f'{g}_bn{i}']).reshape(1, 64)
            pr = p[f'{g}_pr{i}'].reshape(1, 64)
            act, stats = _dense_call(h[:, :64] if i == 0 else h, seg, seg80,
                                     wst, wnt, b, pr)
            h, gm = _norm_call(act, stats, p[f'{g}_g{i}'].reshape(1, 64),
                               p[f'{g}_b{i}'].reshape(1, 64))
            gms.append(gm)
        return jnp.concatenate(gms, axis=1)

    gmh = gcn(header_feat, header_edge_index, 'h')
    gmp = gcn(payload_feat, payload_edge_index, 'p')

    hmask_b = jnp.broadcast_to(header_mask.reshape(B * P, 1), (B * P, D))
    pmask_b = jnp.broadcast_to(payload_mask.reshape(B * P, 1), (B * P, D))

    prow = jnp.tile(jnp.arange(P), B)
    args = []
    for f in ['f1', 'f2']:
        args += [p[f + '_W1'].T, p[f + '_b1'].reshape(1, D),
                 jnp.broadcast_to(p[f + '_pr'][prow][:, None], (B * P, D)),
                 p[f + '_W2'].T, p[f + '_b2'].reshape(1, D)]
    for l in range(2):
        for d in ['f', 'b']:
            args += [p[f'lstm_Wih{l}{d}'].T, p[f'lstm_Whh{l}{d}'].T,
                     p[f'lstm_b{l}{d}'].reshape(1, 2048)]
    args += [p['enc_W'].T, p['enc_b'].reshape(1, D), p['enc_pr'].reshape(1, D)]

    return _tail_call(gmh, gmp, hmask_b, pmask_b, args)
